# Initial kernel scaffold; baseline (speedup 1.0000x reference)
import jax
import jax.numpy as jnp
from jax.experimental import pallas as pl


def kernel(x, edge_index, W1, b1, W2, b2, W_ih, W_hh, b_ih, b_hh, Wp, bp):
    raise NotImplementedError("write your pallas kernel here")


# speedup vs baseline: 17.2883x; 17.2883x over previous
import functools

import jax
import jax.numpy as jnp
from jax import lax
from jax.experimental import pallas as pl
from jax.experimental.pallas import tpu as pltpu
from jax.experimental.pallas import tpu_sc as plsc

N = 10000
E = 320000
D_IN = 128
H = 64

NC = 2
NS = 16
NW = NC * NS
BLK = 128
NBLK = 80
EPW = NBLK * BLK
E_PAD = NW * EPW
N_ACC = 10240
RPT = N_ACC // NS
ZR = 128
NBUF = 4
DUMMY = N_ACC - 1

RB = 2000
GRID = N // RB


def _sc_mesh():
    return plsc.VectorSubcoreMesh(core_axis_name="c", subcore_axis_name="s")


def _sc_degree(dst3):
    @functools.partial(
        pl.kernel,
        mesh=_sc_mesh(),
        out_type=jax.ShapeDtypeStruct((NC, N_ACC), jnp.float32),
        scratch_types=[
            pltpu.VMEM((NBLK, BLK), jnp.int32),
            pltpu.VMEM((BLK,), jnp.float32),
            pltpu.VMEM((RPT,), jnp.float32),
            pltpu.VMEM_SHARED((N_ACC,), jnp.float32),
        ],
        compiler_params=pltpu.CompilerParams(use_tc_tiling_on_sc=False),
    )
    def k(dst_hbm, outp, dstv, ones_v, zv, dacc):
        c = lax.axis_index("c")
        s = lax.axis_index("s")
        wid = s * NC + c
        one16 = jnp.ones((16,), jnp.float32)
        zero16 = jnp.zeros((16,), jnp.float32)
        for i in range(BLK // 16):
            ones_v[pl.ds(i * 16, 16)] = one16

        def zb(i, carry):
            zv[pl.ds(i * 16, 16)] = zero16
            return carry

        lax.fori_loop(0, RPT // 16, zb, 0)
        pltpu.sync_copy(zv, dacc.at[pl.ds(s * RPT, RPT)])
        plsc.subcore_barrier()
        pltpu.sync_copy(dst_hbm.at[wid], dstv)

        def step(j, carry):
            pltpu.sync_copy(ones_v, dacc.at[dstv.at[j]], add=True)
            return carry

        lax.fori_loop(0, NBLK, step, 0)
        plsc.subcore_barrier()
        pltpu.sync_copy(dacc.at[pl.ds(s * RPT, RPT)],
                        outp.at[c, pl.ds(s * RPT, RPT)])

    return k(dst3)


def _sc_scatter_rows(tab, src3, dst3):
    @functools.partial(
        pl.kernel,
        mesh=_sc_mesh(),
        out_type=jax.ShapeDtypeStruct((NC, N_ACC, H), jnp.float32),
        scratch_types=[
            pltpu.VMEM((NBLK, BLK), jnp.int32),
            pltpu.VMEM((NBLK, BLK), jnp.int32),
            [pltpu.VMEM((BLK, H), jnp.float32) for _ in range(NBUF)],
            pltpu.VMEM((ZR, H), jnp.float32),
            pltpu.VMEM_SHARED((N_ACC, H), jnp.float32),
            [pltpu.SemaphoreType.DMA for _ in range(NBUF)],
        ],
        compiler_params=pltpu.CompilerParams(use_tc_tiling_on_sc=False),
    )
    def k(tab_hbm, src_hbm, dst_hbm, outp, srcv, dstv, rows, zrow, acc, sems):
        c = lax.axis_index("c")
        s = lax.axis_index("s")
        wid = s * NC + c
        zero16 = jnp.zeros((16,), jnp.float32)

        def zr_body(r, carry):
            for cc in range(H // 16):
                zrow[r, pl.ds(cc * 16, 16)] = zero16
            return carry

        lax.fori_loop(0, ZR, zr_body, 0)
        for kk in range(RPT // ZR):
            pltpu.sync_copy(zrow, acc.at[pl.ds(s * RPT + kk * ZR, ZR)])
        plsc.subcore_barrier()
        pltpu.sync_copy(src_hbm.at[wid], srcv)
        pltpu.sync_copy(dst_hbm.at[wid], dstv)

        for b in range(NBUF):
            pltpu.async_copy(tab_hbm.at[srcv.at[b]], rows[b], sems[b])

        def outer(g, carry):
            for b in range(NBUF):
                j = g * NBUF + b
                pltpu.make_async_copy(tab_hbm.at[srcv.at[j]],
                                      rows[b], sems[b]).wait()
                pltpu.sync_copy(rows[b], acc.at[dstv.at[j]], add=True)

                @pl.when(j + NBUF < NBLK)
                def _():
                    pltpu.async_copy(tab_hbm.at[srcv.at[j + NBUF]],
                                     rows[b], sems[b])
            return carry

        lax.fori_loop(0, NBLK // NBUF, outer, 0)
        plsc.subcore_barrier()
        pltpu.sync_copy(acc.at[pl.ds(s * RPT, RPT)],
                        outp.at[c, pl.ds(s * RPT, RPT)])

    return k(tab, src3, dst3)


def _tc1_body(x_ref, w1_ref, degp_ref, g1s_ref, dinv_ref):
    deg = degp_ref[:, 0:1] + degp_ref[:, 1:2] + 1.0
    dinv = lax.rsqrt(deg)
    g1 = jnp.dot(x_ref[...], w1_ref[...], preferred_element_type=jnp.float32)
    g1s_ref[...] = g1 * dinv
    dinv_ref[...] = dinv


def _tc1(x, W1, degp_t):
    return pl.pallas_call(
        _tc1_body,
        grid=(GRID,),
        in_specs=[
            pl.BlockSpec((RB, D_IN), lambda i: (i, 0)),
            pl.BlockSpec((D_IN, H), lambda i: (0, 0)),
            pl.BlockSpec((RB, NC), lambda i: (i, 0)),
        ],
        out_specs=[
            pl.BlockSpec((RB, H), lambda i: (i, 0)),
            pl.BlockSpec((RB, 1), lambda i: (i, 0)),
        ],
        out_shape=[
            jax.ShapeDtypeStruct((N, H), jnp.float32),
            jax.ShapeDtypeStruct((N, 1), jnp.float32),
        ],
    )(x, W1, degp_t)


def _tc2_body(acc_ref, gs_ref, dinv_ref, b_ref, w_ref, out_ref):
    s = acc_ref[0] + acc_ref[1]
    dinv = dinv_ref[...]
    h = jnp.maximum(dinv * s + gs_ref[...] * dinv + b_ref[...], 0.0)
    g = jnp.dot(h, w_ref[...], preferred_element_type=jnp.float32)
    out_ref[...] = g * dinv


def _tc2(acc, gs, dinv, b, W):
    return pl.pallas_call(
        _tc2_body,
        grid=(GRID,),
        in_specs=[
            pl.BlockSpec((NC, RB, H), lambda i: (0, i, 0)),
            pl.BlockSpec((RB, H), lambda i: (i, 0)),
            pl.BlockSpec((RB, 1), lambda i: (i, 0)),
            pl.BlockSpec((1, H), lambda i: (0, 0)),
            pl.BlockSpec((H, H), lambda i: (0, 0)),
        ],
        out_specs=pl.BlockSpec((RB, H), lambda i: (i, 0)),
        out_shape=jax.ShapeDtypeStruct((N, H), jnp.float32),
    )(acc, gs, dinv, b, W)


def _tc3_body(acc_ref, gs_ref, dinv_ref, b2_ref, wr_ref, wz_ref, wn_ref,
              bih_ref, bhh_ref, wp_ref, bp_ref, out_ref, hnew_ref):
    s = acc_ref[0] + acc_ref[1]
    dinv = dinv_ref[...]
    h2 = jnp.maximum(dinv * s + gs_ref[...] * dinv + b2_ref[...], 0.0)
    gr = jnp.dot(h2, wr_ref[...], preferred_element_type=jnp.float32)
    gz = jnp.dot(h2, wz_ref[...], preferred_element_type=jnp.float32)
    gn = jnp.dot(h2, wn_ref[...], preferred_element_type=jnp.float32)
    bih = bih_ref[...]
    bhh = bhh_ref[...]
    r = jax.nn.sigmoid(gr + bih[0:1, :] + bhh[0:1, :])
    z = jax.nn.sigmoid(gz + bih[1:2, :] + bhh[1:2, :])
    n = jnp.tanh(gn + bih[2:3, :] + r * bhh[2:3, :])
    hnew = (1.0 - z) * n
    hnew_ref[...] = hnew
    out_ref[...] = (jnp.dot(hnew, wp_ref[...],
                            preferred_element_type=jnp.float32) + bp_ref[...])


def _tc3(acc, gs, dinv, b2, WrT, WzT, WnT, bih3, bhh3, Wp, bp):
    return pl.pallas_call(
        _tc3_body,
        grid=(GRID,),
        in_specs=[
            pl.BlockSpec((NC, RB, H), lambda i: (0, i, 0)),
            pl.BlockSpec((RB, H), lambda i: (i, 0)),
            pl.BlockSpec((RB, 1), lambda i: (i, 0)),
            pl.BlockSpec((1, H), lambda i: (0, 0)),
            pl.BlockSpec((H, H), lambda i: (0, 0)),
            pl.BlockSpec((H, H), lambda i: (0, 0)),
            pl.BlockSpec((H, H), lambda i: (0, 0)),
            pl.BlockSpec((3, H), lambda i: (0, 0)),
            pl.BlockSpec((3, H), lambda i: (0, 0)),
            pl.BlockSpec((H, H), lambda i: (0, 0)),
            pl.BlockSpec((1, H), lambda i: (0, 0)),
        ],
        out_specs=[
            pl.BlockSpec((RB, H), lambda i: (i, 0)),
            pl.BlockSpec((RB, H), lambda i: (i, 0)),
        ],
        out_shape=[
            jax.ShapeDtypeStruct((N, H), jnp.float32),
            jax.ShapeDtypeStruct((N, H), jnp.float32),
        ],
    )(acc, gs, dinv, b2, WrT, WzT, WnT, bih3, bhh3, Wp, bp)


def kernel(x, edge_index, W1, b1, W2, b2, W_ih, W_hh, b_ih, b_hh, Wp, bp):
    pad = E_PAD - E
    src3 = jnp.concatenate(
        [edge_index[0], jnp.zeros((pad,), jnp.int32)]).reshape(NW, NBLK, BLK)
    dst3 = jnp.concatenate(
        [edge_index[1], jnp.full((pad,), DUMMY, jnp.int32)]).reshape(NW, NBLK, BLK)

    degp = _sc_degree(dst3)
    degp_t = degp.T[:N]

    g1s, dinv = _tc1(x, W1, degp_t)
    acc1 = _sc_scatter_rows(g1s, src3, dst3)
    g2s = _tc2(acc1, g1s, dinv, b1.reshape(1, H), W2)
    acc2 = _sc_scatter_rows(g2s, src3, dst3)

    WihT = W_ih.T
    out, hnew = _tc3(
        acc2, g2s, dinv, b2.reshape(1, H),
        WihT[:, 0:H], WihT[:, H:2 * H], WihT[:, 2 * H:3 * H],
        b_ih.reshape(3, H), b_hh.reshape(3, H), Wp, bp.reshape(1, H))
    return (out, hnew[None, :, :])


# speedup vs baseline: 35.3474x; 2.0446x over previous
import functools

import jax
import jax.numpy as jnp
from jax import lax
from jax.experimental import pallas as pl
from jax.experimental.pallas import tpu as pltpu
from jax.experimental.pallas import tpu_sc as plsc

N = 10000
E = 320000
D_IN = 128
H = 64

NC = 2
NS = 16
NW = NC * NS
BLK = 128
NBLK = 80
EPW = NBLK * BLK
E_PAD = NW * EPW
N_ACC = 10240
RPT = N_ACC // NS
ZR = BLK
NBUF = 2
TPT = N // NS
DUMMY = N_ACC - 1

RB = 2000
GRID = N // RB


def _sc_mesh():
    return plsc.VectorSubcoreMesh(core_axis_name="c", subcore_axis_name="s")


def _sc_degree(dst3):
    @functools.partial(
        pl.kernel,
        mesh=_sc_mesh(),
        out_type=jax.ShapeDtypeStruct((NC, N_ACC), jnp.float32),
        scratch_types=[
            pltpu.VMEM((NBLK, BLK), jnp.int32),
            pltpu.VMEM((BLK,), jnp.float32),
            pltpu.VMEM((RPT,), jnp.float32),
            pltpu.VMEM_SHARED((N_ACC,), jnp.float32),
        ],
        compiler_params=pltpu.CompilerParams(use_tc_tiling_on_sc=False),
    )
    def k(dst_hbm, outp, dstv, ones_v, zv, dacc):
        c = lax.axis_index("c")
        s = lax.axis_index("s")
        wid = s * NC + c
        one16 = jnp.ones((16,), jnp.float32)
        zero16 = jnp.zeros((16,), jnp.float32)
        for i in range(BLK // 16):
            ones_v[pl.ds(i * 16, 16)] = one16

        def zb(i, carry):
            zv[pl.ds(i * 16, 16)] = zero16
            return carry

        lax.fori_loop(0, RPT // 16, zb, 0)
        pltpu.sync_copy(zv, dacc.at[pl.ds(s * RPT, RPT)])
        plsc.subcore_barrier()
        pltpu.sync_copy(dst_hbm.at[wid], dstv)

        def step(j, carry):
            pltpu.sync_copy(ones_v, dacc.at[dstv.at[j]], add=True)
            return carry

        lax.fori_loop(0, NBLK, step, 0)
        plsc.subcore_barrier()
        pltpu.sync_copy(dacc.at[pl.ds(s * RPT, RPT)],
                        outp.at[c, pl.ds(s * RPT, RPT)])

    return k(dst3)


def _sc_scatter_rows(tab, src3, dst3):
    @functools.partial(
        pl.kernel,
        mesh=_sc_mesh(),
        out_type=jax.ShapeDtypeStruct((NC, N_ACC, H), jnp.float32),
        scratch_types=[
            pltpu.VMEM((NBLK, BLK), jnp.int32),
            pltpu.VMEM((NBLK, BLK), jnp.int32),
            [pltpu.VMEM((BLK, H), jnp.float32) for _ in range(NBUF)],
            pltpu.VMEM_SHARED((N_ACC, H), jnp.float32),
            pltpu.VMEM_SHARED((N, H), jnp.float32),
            [pltpu.SemaphoreType.DMA for _ in range(NBUF)],
        ],
        compiler_params=pltpu.CompilerParams(use_tc_tiling_on_sc=False),
    )
    def k(tab_hbm, src_hbm, dst_hbm, outp, srcv, dstv, rows, acc, tabs, sems):
        c = lax.axis_index("c")
        s = lax.axis_index("s")
        wid = s * NC + c
        zero16 = jnp.zeros((16,), jnp.float32)

        pltpu.sync_copy(tab_hbm.at[pl.ds(s * TPT, TPT)],
                        tabs.at[pl.ds(s * TPT, TPT)])

        def zr_body(r, carry):
            for cc in range(H // 16):
                rows[0][r, pl.ds(cc * 16, 16)] = zero16
            return carry

        lax.fori_loop(0, ZR, zr_body, 0)
        for kk in range(RPT // ZR):
            pltpu.sync_copy(rows[0], acc.at[pl.ds(s * RPT + kk * ZR, ZR)])
        pltpu.sync_copy(src_hbm.at[wid], srcv)
        pltpu.sync_copy(dst_hbm.at[wid], dstv)
        plsc.subcore_barrier()

        for b in range(NBUF):
            pltpu.async_copy(tabs.at[srcv.at[b]], rows[b], sems[b])

        def outer(g, carry):
            for b in range(NBUF):
                j = g * NBUF + b
                pltpu.make_async_copy(tabs.at[srcv.at[j]],
                                      rows[b], sems[b]).wait()
                pltpu.sync_copy(rows[b], acc.at[dstv.at[j]], add=True)

                @pl.when(j + NBUF < NBLK)
                def _():
                    pltpu.async_copy(tabs.at[srcv.at[j + NBUF]],
                                     rows[b], sems[b])
            return carry

        lax.fori_loop(0, NBLK // NBUF, outer, 0)
        plsc.subcore_barrier()
        pltpu.sync_copy(acc.at[pl.ds(s * RPT, RPT)],
                        outp.at[c, pl.ds(s * RPT, RPT)])

    return k(tab, src3, dst3)


def _tc1_body(x_ref, w1_ref, degp_ref, g1s_ref, dinv_ref):
    deg = degp_ref[:, 0:1] + degp_ref[:, 1:2] + 1.0
    dinv = lax.rsqrt(deg)
    g1 = jnp.dot(x_ref[...], w1_ref[...], preferred_element_type=jnp.float32)
    g1s_ref[...] = g1 * dinv
    dinv_ref[...] = dinv


def _tc1(x, W1, degp_t):
    return pl.pallas_call(
        _tc1_body,
        grid=(GRID,),
        in_specs=[
            pl.BlockSpec((RB, D_IN), lambda i: (i, 0)),
            pl.BlockSpec((D_IN, H), lambda i: (0, 0)),
            pl.BlockSpec((RB, NC), lambda i: (i, 0)),
        ],
        out_specs=[
            pl.BlockSpec((RB, H), lambda i: (i, 0)),
            pl.BlockSpec((RB, 1), lambda i: (i, 0)),
        ],
        out_shape=[
            jax.ShapeDtypeStruct((N, H), jnp.float32),
            jax.ShapeDtypeStruct((N, 1), jnp.float32),
        ],
    )(x, W1, degp_t)


def _tc2_body(acc_ref, gs_ref, dinv_ref, b_ref, w_ref, out_ref):
    s = acc_ref[0] + acc_ref[1]
    dinv = dinv_ref[...]
    h = jnp.maximum(dinv * s + gs_ref[...] * dinv + b_ref[...], 0.0)
    g = jnp.dot(h, w_ref[...], preferred_element_type=jnp.float32)
    out_ref[...] = g * dinv


def _tc2(acc, gs, dinv, b, W):
    return pl.pallas_call(
        _tc2_body,
        grid=(GRID,),
        in_specs=[
            pl.BlockSpec((NC, RB, H), lambda i: (0, i, 0)),
            pl.BlockSpec((RB, H), lambda i: (i, 0)),
            pl.BlockSpec((RB, 1), lambda i: (i, 0)),
            pl.BlockSpec((1, H), lambda i: (0, 0)),
            pl.BlockSpec((H, H), lambda i: (0, 0)),
        ],
        out_specs=pl.BlockSpec((RB, H), lambda i: (i, 0)),
        out_shape=jax.ShapeDtypeStruct((N, H), jnp.float32),
    )(acc, gs, dinv, b, W)


def _tc3_body(acc_ref, gs_ref, dinv_ref, b2_ref, wr_ref, wz_ref, wn_ref,
              bih_ref, bhh_ref, wp_ref, bp_ref, out_ref, hnew_ref):
    s = acc_ref[0] + acc_ref[1]
    dinv = dinv_ref[...]
    h2 = jnp.maximum(dinv * s + gs_ref[...] * dinv + b2_ref[...], 0.0)
    gr = jnp.dot(h2, wr_ref[...], preferred_element_type=jnp.float32)
    gz = jnp.dot(h2, wz_ref[...], preferred_element_type=jnp.float32)
    gn = jnp.dot(h2, wn_ref[...], preferred_element_type=jnp.float32)
    bih = bih_ref[...]
    bhh = bhh_ref[...]
    r = jax.nn.sigmoid(gr + bih[0:1, :] + bhh[0:1, :])
    z = jax.nn.sigmoid(gz + bih[1:2, :] + bhh[1:2, :])
    n = jnp.tanh(gn + bih[2:3, :] + r * bhh[2:3, :])
    hnew = (1.0 - z) * n
    hnew_ref[...] = hnew
    out_ref[...] = (jnp.dot(hnew, wp_ref[...],
                            preferred_element_type=jnp.float32) + bp_ref[...])


def _tc3(acc, gs, dinv, b2, WrT, WzT, WnT, bih3, bhh3, Wp, bp):
    return pl.pallas_call(
        _tc3_body,
        grid=(GRID,),
        in_specs=[
            pl.BlockSpec((NC, RB, H), lambda i: (0, i, 0)),
            pl.BlockSpec((RB, H), lambda i: (i, 0)),
            pl.BlockSpec((RB, 1), lambda i: (i, 0)),
            pl.BlockSpec((1, H), lambda i: (0, 0)),
            pl.BlockSpec((H, H), lambda i: (0, 0)),
            pl.BlockSpec((H, H), lambda i: (0, 0)),
            pl.BlockSpec((H, H), lambda i: (0, 0)),
            pl.BlockSpec((3, H), lambda i: (0, 0)),
            pl.BlockSpec((3, H), lambda i: (0, 0)),
            pl.BlockSpec((H, H), lambda i: (0, 0)),
            pl.BlockSpec((1, H), lambda i: (0, 0)),
        ],
        out_specs=[
            pl.BlockSpec((RB, H), lambda i: (i, 0)),
            pl.BlockSpec((RB, H), lambda i: (i, 0)),
        ],
        out_shape=[
            jax.ShapeDtypeStruct((N, H), jnp.float32),
            jax.ShapeDtypeStruct((N, H), jnp.float32),
        ],
    )(acc, gs, dinv, b2, WrT, WzT, WnT, bih3, bhh3, Wp, bp)


def kernel(x, edge_index, W1, b1, W2, b2, W_ih, W_hh, b_ih, b_hh, Wp, bp):
    pad = E_PAD - E
    src3 = jnp.concatenate(
        [edge_index[0], jnp.zeros((pad,), jnp.int32)]).reshape(NW, NBLK, BLK)
    dst3 = jnp.concatenate(
        [edge_index[1], jnp.full((pad,), DUMMY, jnp.int32)]).reshape(NW, NBLK, BLK)

    degp = _sc_degree(dst3)
    degp_t = degp.T[:N]

    g1s, dinv = _tc1(x, W1, degp_t)
    acc1 = _sc_scatter_rows(g1s, src3, dst3)
    g2s = _tc2(acc1, g1s, dinv, b1.reshape(1, H), W2)
    acc2 = _sc_scatter_rows(g2s, src3, dst3)

    WihT = W_ih.T
    out, hnew = _tc3(
        acc2, g2s, dinv, b2.reshape(1, H),
        WihT[:, 0:H], WihT[:, H:2 * H], WihT[:, 2 * H:3 * H],
        b_ih.reshape(3, H), b_hh.reshape(3, H), Wp, bp.reshape(1, H))
    return (out, hnew[None, :, :])


# speedup vs baseline: 35.6441x; 1.0084x over previous
import functools

import jax
import jax.numpy as jnp
from jax import lax
from jax.experimental import pallas as pl
from jax.experimental.pallas import tpu as pltpu
from jax.experimental.pallas import tpu_sc as plsc

N = 10000
E = 320000
D_IN = 128
H = 64

NC = 2
NS = 16
NW = NC * NS
BLK = 128
NBLK = 80
EPW = NBLK * BLK
E_PAD = NW * EPW
N_ACC = 10240
RPT = N_ACC // NS
ZR = BLK
NBUF = 2
TPT = N // NS
DUMMY = N_ACC - 1

RB = 2000
GRID = N // RB


def _sc_mesh():
    return plsc.VectorSubcoreMesh(core_axis_name="c", subcore_axis_name="s")


def _sc_degree(dst3):
    @functools.partial(
        pl.kernel,
        mesh=_sc_mesh(),
        out_type=jax.ShapeDtypeStruct((NC, N_ACC), jnp.float32),
        scratch_types=[
            pltpu.VMEM((NBLK, BLK), jnp.int32),
            pltpu.VMEM((BLK,), jnp.float32),
            pltpu.VMEM((RPT,), jnp.float32),
            pltpu.VMEM_SHARED((N_ACC,), jnp.float32),
        ],
        compiler_params=pltpu.CompilerParams(use_tc_tiling_on_sc=False),
    )
    def k(dst_hbm, outp, dstv, ones_v, zv, dacc):
        c = lax.axis_index("c")
        s = lax.axis_index("s")
        wid = s * NC + c
        one16 = jnp.ones((16,), jnp.float32)
        zero16 = jnp.zeros((16,), jnp.float32)
        for i in range(BLK // 16):
            ones_v[pl.ds(i * 16, 16)] = one16

        def zb(i, carry):
            zv[pl.ds(i * 16, 16)] = zero16
            return carry

        lax.fori_loop(0, RPT // 16, zb, 0)
        pltpu.sync_copy(zv, dacc.at[pl.ds(s * RPT, RPT)])
        plsc.subcore_barrier()
        pltpu.sync_copy(dst_hbm.at[wid], dstv)

        def step(j, carry):
            pltpu.sync_copy(ones_v, dacc.at[dstv.at[j]], add=True)
            return carry

        lax.fori_loop(0, NBLK, step, 0)
        plsc.subcore_barrier()
        pltpu.sync_copy(dacc.at[pl.ds(s * RPT, RPT)],
                        outp.at[c, pl.ds(s * RPT, RPT)])

    return k(dst3)


def _sc_scatter_rows(tab, src3, dst3):
    @functools.partial(
        pl.kernel,
        mesh=_sc_mesh(),
        out_type=[jax.ShapeDtypeStruct((N_ACC, H), jnp.float32),
                  jax.ShapeDtypeStruct((N_ACC, H), jnp.float32)],
        scratch_types=[
            pltpu.VMEM((NBLK, BLK), jnp.int32),
            pltpu.VMEM((NBLK, BLK), jnp.int32),
            [pltpu.VMEM((BLK, H), jnp.float32) for _ in range(NBUF)],
            pltpu.VMEM_SHARED((N_ACC, H), jnp.float32),
            pltpu.VMEM_SHARED((N, H), jnp.float32),
            [pltpu.SemaphoreType.DMA for _ in range(NBUF)],
            [pltpu.SemaphoreType.DMA for _ in range(NBUF)],
        ],
        compiler_params=pltpu.CompilerParams(use_tc_tiling_on_sc=False),
    )
    def k(tab_hbm, src_hbm, dst_hbm, out0, out1, srcv, dstv, rows, acc, tabs,
          sems, ssems):
        c = lax.axis_index("c")
        s = lax.axis_index("s")
        wid = s * NC + c
        zero16 = jnp.zeros((16,), jnp.float32)

        pltpu.sync_copy(tab_hbm.at[pl.ds(s * TPT, TPT)],
                        tabs.at[pl.ds(s * TPT, TPT)])

        def zr_body(r, carry):
            for cc in range(H // 16):
                rows[0][r, pl.ds(cc * 16, 16)] = zero16
            return carry

        lax.fori_loop(0, ZR, zr_body, 0)
        for kk in range(RPT // ZR):
            pltpu.sync_copy(rows[0], acc.at[pl.ds(s * RPT + kk * ZR, ZR)])
        pltpu.sync_copy(src_hbm.at[wid], srcv)
        pltpu.sync_copy(dst_hbm.at[wid], dstv)
        plsc.subcore_barrier()

        for b in range(NBUF):
            pltpu.async_copy(tabs.at[srcv.at[b]], rows[b], sems[b])

        def outer(g, carry):
            for b in range(NBUF):
                j = g * NBUF + b
                pltpu.make_async_copy(tabs.at[srcv.at[j]],
                                      rows[b], sems[b]).wait()
                pltpu.async_copy(rows[b], acc.at[dstv.at[j]], ssems[b],
                                 add=True)

                @pl.when(j + NBUF < NBLK)
                def _():
                    pltpu.make_async_copy(rows[b], acc.at[dstv.at[j]],
                                          ssems[b]).wait()
                    pltpu.async_copy(tabs.at[srcv.at[j + NBUF]],
                                     rows[b], sems[b])
            return carry

        lax.fori_loop(0, NBLK // NBUF, outer, 0)
        for b in range(NBUF):
            j = NBLK - NBUF + b
            pltpu.make_async_copy(rows[b], acc.at[dstv.at[j]],
                                  ssems[b]).wait()
        plsc.subcore_barrier()

        @pl.when(c == 0)
        def _():
            pltpu.sync_copy(acc.at[pl.ds(s * RPT, RPT)],
                            out0.at[pl.ds(s * RPT, RPT)])

        @pl.when(c == 1)
        def _():
            pltpu.sync_copy(acc.at[pl.ds(s * RPT, RPT)],
                            out1.at[pl.ds(s * RPT, RPT)])

    return k(tab, src3, dst3)


def _tc1_body(x_ref, w1_ref, degp_ref, g1s_ref, dinv_ref):
    deg = degp_ref[:, 0:1] + degp_ref[:, 1:2] + 1.0
    dinv = lax.rsqrt(deg)
    g1 = jnp.dot(x_ref[...], w1_ref[...], preferred_element_type=jnp.float32)
    g1s_ref[...] = g1 * dinv
    dinv_ref[...] = dinv


def _tc1(x, W1, degp):
    return pl.pallas_call(
        _tc1_body,
        grid=(GRID,),
        in_specs=[
            pl.BlockSpec((RB, D_IN), lambda i: (i, 0)),
            pl.BlockSpec((D_IN, H), lambda i: (0, 0)),
            pl.BlockSpec((RB, NC), lambda i: (i, 0)),
        ],
        out_specs=[
            pl.BlockSpec((RB, H), lambda i: (i, 0)),
            pl.BlockSpec((RB, 1), lambda i: (i, 0)),
        ],
        out_shape=[
            jax.ShapeDtypeStruct((N, H), jnp.float32),
            jax.ShapeDtypeStruct((N, 1), jnp.float32),
        ],
    )(x, W1, degp)


def _tc2_body(a0_ref, a1_ref, gs_ref, dinv_ref, b_ref, w_ref, out_ref):
    s = a0_ref[...] + a1_ref[...]
    dinv = dinv_ref[...]
    h = jnp.maximum(dinv * s + gs_ref[...] * dinv + b_ref[...], 0.0)
    g = jnp.dot(h, w_ref[...], preferred_element_type=jnp.float32)
    out_ref[...] = g * dinv


def _tc2(a0, a1, gs, dinv, b, W):
    return pl.pallas_call(
        _tc2_body,
        grid=(GRID,),
        in_specs=[
            pl.BlockSpec((RB, H), lambda i: (i, 0)),
            pl.BlockSpec((RB, H), lambda i: (i, 0)),
            pl.BlockSpec((RB, H), lambda i: (i, 0)),
            pl.BlockSpec((RB, 1), lambda i: (i, 0)),
            pl.BlockSpec((1, H), lambda i: (0, 0)),
            pl.BlockSpec((H, H), lambda i: (0, 0)),
        ],
        out_specs=pl.BlockSpec((RB, H), lambda i: (i, 0)),
        out_shape=jax.ShapeDtypeStruct((N, H), jnp.float32),
    )(a0, a1, gs, dinv, b, W)


def _tc3_body(a0_ref, a1_ref, gs_ref, dinv_ref, b2_ref, wr_ref, wz_ref,
              wn_ref, bih_ref, bhh_ref, wp_ref, bp_ref, out_ref, hnew_ref):
    s = a0_ref[...] + a1_ref[...]
    dinv = dinv_ref[...]
    h2 = jnp.maximum(dinv * s + gs_ref[...] * dinv + b2_ref[...], 0.0)
    gr = jnp.dot(h2, wr_ref[...], preferred_element_type=jnp.float32)
    gz = jnp.dot(h2, wz_ref[...], preferred_element_type=jnp.float32)
    gn = jnp.dot(h2, wn_ref[...], preferred_element_type=jnp.float32)
    bih = bih_ref[...]
    bhh = bhh_ref[...]
    r = jax.nn.sigmoid(gr + bih[0:1, :] + bhh[0:1, :])
    z = jax.nn.sigmoid(gz + bih[1:2, :] + bhh[1:2, :])
    n = jnp.tanh(gn + bih[2:3, :] + r * bhh[2:3, :])
    hnew = (1.0 - z) * n
    hnew_ref[0] = hnew
    out_ref[...] = (jnp.dot(hnew, wp_ref[...],
                            preferred_element_type=jnp.float32) + bp_ref[...])


def _tc3(a0, a1, gs, dinv, b2, WrT, WzT, WnT, bih3, bhh3, Wp, bp):
    return pl.pallas_call(
        _tc3_body,
        grid=(GRID,),
        in_specs=[
            pl.BlockSpec((RB, H), lambda i: (i, 0)),
            pl.BlockSpec((RB, H), lambda i: (i, 0)),
            pl.BlockSpec((RB, H), lambda i: (i, 0)),
            pl.BlockSpec((RB, 1), lambda i: (i, 0)),
            pl.BlockSpec((1, H), lambda i: (0, 0)),
            pl.BlockSpec((H, H), lambda i: (0, 0)),
            pl.BlockSpec((H, H), lambda i: (0, 0)),
            pl.BlockSpec((H, H), lambda i: (0, 0)),
            pl.BlockSpec((3, H), lambda i: (0, 0)),
            pl.BlockSpec((3, H), lambda i: (0, 0)),
            pl.BlockSpec((H, H), lambda i: (0, 0)),
            pl.BlockSpec((1, H), lambda i: (0, 0)),
        ],
        out_specs=[
            pl.BlockSpec((RB, H), lambda i: (i, 0)),
            pl.BlockSpec((1, RB, H), lambda i: (0, i, 0)),
        ],
        out_shape=[
            jax.ShapeDtypeStruct((N, H), jnp.float32),
            jax.ShapeDtypeStruct((1, N, H), jnp.float32),
        ],
    )(a0, a1, gs, dinv, b2, WrT, WzT, WnT, bih3, bhh3, Wp, bp)


def kernel(x, edge_index, W1, b1, W2, b2, W_ih, W_hh, b_ih, b_hh, Wp, bp):
    pad = E_PAD - E
    src3 = jnp.concatenate(
        [edge_index[0], jnp.zeros((pad,), jnp.int32)]).reshape(NW, NBLK, BLK)
    dst3 = jnp.concatenate(
        [edge_index[1], jnp.full((pad,), DUMMY, jnp.int32)]).reshape(NW, NBLK, BLK)

    degp = _sc_degree(dst3)
    degp_t = degp.T[:N]

    g1s, dinv = _tc1(x, W1, degp_t)
    a10, a11 = _sc_scatter_rows(g1s, src3, dst3)
    g2s = _tc2(a10, a11, g1s, dinv, b1.reshape(1, H), W2)
    a20, a21 = _sc_scatter_rows(g2s, src3, dst3)

    WihT = W_ih.T
    out, hidden = _tc3(
        a20, a21, g2s, dinv, b2.reshape(1, H),
        WihT[:, 0:H], WihT[:, H:2 * H], WihT[:, 2 * H:3 * H],
        b_ih.reshape(3, H), b_hh.reshape(3, H), Wp, bp.reshape(1, H))
    return (out, hidden)


# speedup vs baseline: 46.3292x; 1.2998x over previous
import functools

import jax
import jax.numpy as jnp
from jax import lax
from jax.experimental import pallas as pl
from jax.experimental.pallas import tpu as pltpu
from jax.experimental.pallas import tpu_sc as plsc

N = 10000
E = 320000
D_IN = 128
H = 64

NC = 2
NS = 16
NW = NC * NS
BLK = 128
NBLK = 80
EPW = NBLK * BLK
E_PAD = NW * EPW
N_ACC = 10240
RPT = N_ACC // NS
ZR = BLK
NBUF = 4
TPT = N // NS
DUMMY = N_ACC - 1

RB = 1000
GRID = N // RB


def _sc_mesh():
    return plsc.VectorSubcoreMesh(core_axis_name="c", subcore_axis_name="s")


def _sc_degree(dst3):
    @functools.partial(
        pl.kernel,
        mesh=_sc_mesh(),
        out_type=jax.ShapeDtypeStruct((NC, N_ACC), jnp.float32),
        scratch_types=[
            pltpu.VMEM((NBLK, BLK), jnp.int32),
            pltpu.VMEM((BLK,), jnp.float32),
            pltpu.VMEM((RPT,), jnp.float32),
            pltpu.VMEM_SHARED((N_ACC,), jnp.float32),
        ],
        compiler_params=pltpu.CompilerParams(use_tc_tiling_on_sc=False),
    )
    def k(dst_hbm, outp, dstv, ones_v, zv, dacc):
        c = lax.axis_index("c")
        s = lax.axis_index("s")
        wid = s * NC + c
        one16 = jnp.ones((16,), jnp.float32)
        zero16 = jnp.zeros((16,), jnp.float32)
        for i in range(BLK // 16):
            ones_v[pl.ds(i * 16, 16)] = one16

        def zb(i, carry):
            zv[pl.ds(i * 16, 16)] = zero16
            return carry

        lax.fori_loop(0, RPT // 16, zb, 0)
        pltpu.sync_copy(zv, dacc.at[pl.ds(s * RPT, RPT)])
        plsc.subcore_barrier()
        pltpu.sync_copy(dst_hbm.at[wid], dstv)

        def step(j, carry):
            pltpu.sync_copy(ones_v, dacc.at[dstv.at[j]], add=True)
            return carry

        lax.fori_loop(0, NBLK, step, 0)
        plsc.subcore_barrier()
        pltpu.sync_copy(dacc.at[pl.ds(s * RPT, RPT)],
                        outp.at[c, pl.ds(s * RPT, RPT)])

    return k(dst3)


def _sc_scatter_rows(tab, src3, dst3):
    @functools.partial(
        pl.kernel,
        mesh=_sc_mesh(),
        out_type=[jax.ShapeDtypeStruct((N_ACC, H), jnp.bfloat16),
                  jax.ShapeDtypeStruct((N_ACC, H), jnp.bfloat16)],
        scratch_types=[
            pltpu.VMEM((NBLK, BLK), jnp.int32),
            pltpu.VMEM((NBLK, BLK), jnp.int32),
            [pltpu.VMEM((BLK, H), jnp.bfloat16) for _ in range(NBUF)],
            pltpu.VMEM_SHARED((N_ACC, H), jnp.bfloat16),
            pltpu.VMEM_SHARED((N, H), jnp.bfloat16),
            [pltpu.SemaphoreType.DMA for _ in range(NBUF)],
            [pltpu.SemaphoreType.DMA for _ in range(NBUF)],
        ],
        compiler_params=pltpu.CompilerParams(use_tc_tiling_on_sc=False),
    )
    def k(tab_hbm, src_hbm, dst_hbm, out0, out1, srcv, dstv, rows, acc, tabs,
          sems, ssems):
        c = lax.axis_index("c")
        s = lax.axis_index("s")
        wid = s * NC + c
        zero32 = jnp.zeros((32,), jnp.bfloat16)

        pltpu.sync_copy(tab_hbm.at[pl.ds(s * TPT, TPT)],
                        tabs.at[pl.ds(s * TPT, TPT)])

        def zr_body(r, carry):
            for cc in range(H // 32):
                rows[0][r, pl.ds(cc * 32, 32)] = zero32
            return carry

        lax.fori_loop(0, ZR, zr_body, 0)
        for kk in range(RPT // ZR):
            pltpu.sync_copy(rows[0], acc.at[pl.ds(s * RPT + kk * ZR, ZR)])
        pltpu.sync_copy(src_hbm.at[wid], srcv)
        pltpu.sync_copy(dst_hbm.at[wid], dstv)
        plsc.subcore_barrier()

        for b in range(NBUF):
            pltpu.async_copy(tabs.at[srcv.at[b]], rows[b], sems[b])

        def outer(g, carry):
            for b in range(NBUF):
                j = g * NBUF + b
                pltpu.make_async_copy(tabs.at[srcv.at[j]],
                                      rows[b], sems[b]).wait()
                pltpu.async_copy(rows[b], acc.at[dstv.at[j]], ssems[b],
                                 add=True)

                @pl.when(j + NBUF < NBLK)
                def _():
                    pltpu.make_async_copy(rows[b], acc.at[dstv.at[j]],
                                          ssems[b]).wait()
                    pltpu.async_copy(tabs.at[srcv.at[j + NBUF]],
                                     rows[b], sems[b])
            return carry

        lax.fori_loop(0, NBLK // NBUF, outer, 0)
        for b in range(NBUF):
            j = NBLK - NBUF + b
            pltpu.make_async_copy(rows[b], acc.at[dstv.at[j]],
                                  ssems[b]).wait()
        plsc.subcore_barrier()

        @pl.when(c == 0)
        def _():
            pltpu.sync_copy(acc.at[pl.ds(s * RPT, RPT)],
                            out0.at[pl.ds(s * RPT, RPT)])

        @pl.when(c == 1)
        def _():
            pltpu.sync_copy(acc.at[pl.ds(s * RPT, RPT)],
                            out1.at[pl.ds(s * RPT, RPT)])

    return k(tab, src3, dst3)


def _tc1_body(x_ref, w1_ref, degp_ref, g1s_ref, dinv_ref):
    deg = degp_ref[:, 0:1] + degp_ref[:, 1:2] + 1.0
    dinv = lax.rsqrt(deg)
    g1 = jnp.dot(x_ref[...], w1_ref[...], preferred_element_type=jnp.float32)
    g1s_ref[...] = (g1 * dinv).astype(jnp.bfloat16)
    dinv_ref[...] = dinv


def _tc1(x, W1, degp):
    return pl.pallas_call(
        _tc1_body,
        grid=(GRID,),
        in_specs=[
            pl.BlockSpec((RB, D_IN), lambda i: (i, 0)),
            pl.BlockSpec((D_IN, H), lambda i: (0, 0)),
            pl.BlockSpec((RB, NC), lambda i: (i, 0)),
        ],
        out_specs=[
            pl.BlockSpec((RB, H), lambda i: (i, 0)),
            pl.BlockSpec((RB, 1), lambda i: (i, 0)),
        ],
        out_shape=[
            jax.ShapeDtypeStruct((N, H), jnp.bfloat16),
            jax.ShapeDtypeStruct((N, 1), jnp.float32),
        ],
    )(x, W1, degp)


def _tc2_body(a0_ref, a1_ref, gs_ref, dinv_ref, b_ref, w_ref, out_ref):
    s = (a0_ref[...].astype(jnp.float32) + a1_ref[...].astype(jnp.float32))
    gs = gs_ref[...].astype(jnp.float32)
    dinv = dinv_ref[...]
    h = jnp.maximum(dinv * s + gs * dinv + b_ref[...], 0.0)
    g = jnp.dot(h, w_ref[...], preferred_element_type=jnp.float32)
    out_ref[...] = (g * dinv).astype(jnp.bfloat16)


def _tc2(a0, a1, gs, dinv, b, W):
    return pl.pallas_call(
        _tc2_body,
        grid=(GRID,),
        in_specs=[
            pl.BlockSpec((RB, H), lambda i: (i, 0)),
            pl.BlockSpec((RB, H), lambda i: (i, 0)),
            pl.BlockSpec((RB, H), lambda i: (i, 0)),
            pl.BlockSpec((RB, 1), lambda i: (i, 0)),
            pl.BlockSpec((1, H), lambda i: (0, 0)),
            pl.BlockSpec((H, H), lambda i: (0, 0)),
        ],
        out_specs=pl.BlockSpec((RB, H), lambda i: (i, 0)),
        out_shape=jax.ShapeDtypeStruct((N, H), jnp.bfloat16),
    )(a0, a1, gs, dinv, b, W)


def _tc3_body(a0_ref, a1_ref, gs_ref, dinv_ref, b2_ref, wr_ref, wz_ref,
              wn_ref, bih_ref, bhh_ref, wp_ref, bp_ref, out_ref, hnew_ref):
    s = (a0_ref[...].astype(jnp.float32) + a1_ref[...].astype(jnp.float32))
    gs = gs_ref[...].astype(jnp.float32)
    dinv = dinv_ref[...]
    h2 = jnp.maximum(dinv * s + gs * dinv + b2_ref[...], 0.0)
    gr = jnp.dot(h2, wr_ref[...], preferred_element_type=jnp.float32)
    gz = jnp.dot(h2, wz_ref[...], preferred_element_type=jnp.float32)
    gn = jnp.dot(h2, wn_ref[...], preferred_element_type=jnp.float32)
    bih = bih_ref[...]
    bhh = bhh_ref[...]
    r = jax.nn.sigmoid(gr + bih[0:1, :] + bhh[0:1, :])
    z = jax.nn.sigmoid(gz + bih[1:2, :] + bhh[1:2, :])
    n = jnp.tanh(gn + bih[2:3, :] + r * bhh[2:3, :])
    hnew = (1.0 - z) * n
    hnew_ref[0] = hnew
    out_ref[...] = (jnp.dot(hnew, wp_ref[...],
                            preferred_element_type=jnp.float32) + bp_ref[...])


def _tc3(a0, a1, gs, dinv, b2, WrT, WzT, WnT, bih3, bhh3, Wp, bp):
    return pl.pallas_call(
        _tc3_body,
        grid=(GRID,),
        in_specs=[
            pl.BlockSpec((RB, H), lambda i: (i, 0)),
            pl.BlockSpec((RB, H), lambda i: (i, 0)),
            pl.BlockSpec((RB, H), lambda i: (i, 0)),
            pl.BlockSpec((RB, 1), lambda i: (i, 0)),
            pl.BlockSpec((1, H), lambda i: (0, 0)),
            pl.BlockSpec((H, H), lambda i: (0, 0)),
            pl.BlockSpec((H, H), lambda i: (0, 0)),
            pl.BlockSpec((H, H), lambda i: (0, 0)),
            pl.BlockSpec((3, H), lambda i: (0, 0)),
            pl.BlockSpec((3, H), lambda i: (0, 0)),
            pl.BlockSpec((H, H), lambda i: (0, 0)),
            pl.BlockSpec((1, H), lambda i: (0, 0)),
        ],
        out_specs=[
            pl.BlockSpec((RB, H), lambda i: (i, 0)),
            pl.BlockSpec((1, RB, H), lambda i: (0, i, 0)),
        ],
        out_shape=[
            jax.ShapeDtypeStruct((N, H), jnp.float32),
            jax.ShapeDtypeStruct((1, N, H), jnp.float32),
        ],
    )(a0, a1, gs, dinv, b2, WrT, WzT, WnT, bih3, bhh3, Wp, bp)


def kernel(x, edge_index, W1, b1, W2, b2, W_ih, W_hh, b_ih, b_hh, Wp, bp):
    pad = E_PAD - E
    src3 = jnp.concatenate(
        [edge_index[0], jnp.zeros((pad,), jnp.int32)]).reshape(NW, NBLK, BLK)
    dst3 = jnp.concatenate(
        [edge_index[1], jnp.full((pad,), DUMMY, jnp.int32)]).reshape(NW, NBLK, BLK)

    degp = _sc_degree(dst3)
    degp_t = degp.T[:N]

    g1s, dinv = _tc1(x, W1, degp_t)
    a10, a11 = _sc_scatter_rows(g1s, src3, dst3)
    g2s = _tc2(a10, a11, g1s, dinv, b1.reshape(1, H), W2)
    a20, a21 = _sc_scatter_rows(g2s, src3, dst3)

    WihT = W_ih.T
    out, hidden = _tc3(
        a20, a21, g2s, dinv, b2.reshape(1, H),
        WihT[:, 0:H], WihT[:, H:2 * H], WihT[:, 2 * H:3 * H],
        b_ih.reshape(3, H), b_hh.reshape(3, H), Wp, bp.reshape(1, H))
    return (out, hidden)
